# Initial kernel scaffold; baseline (speedup 1.0000x reference)
import jax
import jax.numpy as jnp
from jax.experimental import pallas as pl


def kernel(x, edge_index, edge_weight, W1, b1, W2, b2, distance_weight):
    raise NotImplementedError("write your pallas kernel here")


# speedup vs baseline: 4.5499x; 4.5499x over previous
import functools

import jax
import jax.numpy as jnp
from jax import lax
from jax.experimental import pallas as pl
from jax.experimental.pallas import tpu as pltpu
from jax.experimental.pallas import tpu_sc as plsc

N = 10000
E = 320000
NFEAT = 128
NHID = 128
NCLASS = 64

HALF = NCLASS // 2
NTILES = 16
N_PAD = 10240
ROWS_PER_TILE = N_PAD // NTILES
CHUNK = 128
NCHUNKS = E // CHUNK
CHUNKS_PER_TILE = -(-NCHUNKS // NTILES)
HOPS = 5


def _mlp_body(x_ref, w1t_ref, b1_ref, w2t_ref, b2_ref, h_ref):
    h1 = jnp.dot(x_ref[...], w1t_ref[...], preferred_element_type=jnp.float32)
    h1 = jnp.maximum(h1 + b1_ref[...], 0.0)
    h2 = jnp.dot(h1, w2t_ref[...], preferred_element_type=jnp.float32)
    h2 = h2 + b2_ref[...]
    h_ref[0] = h2[:, :HALF]
    h_ref[1] = h2[:, HALF:]


def _mlp(x, W1, b1, W2, b2):
    blk = ROWS_PER_TILE
    grid = N_PAD // blk
    return pl.pallas_call(
        _mlp_body,
        grid=(grid,),
        in_specs=[
            pl.BlockSpec((blk, NFEAT), lambda i: (i, 0)),
            pl.BlockSpec((NFEAT, NHID), lambda i: (0, 0)),
            pl.BlockSpec((1, NHID), lambda i: (0, 0)),
            pl.BlockSpec((NHID, NCLASS), lambda i: (0, 0)),
            pl.BlockSpec((1, NCLASS), lambda i: (0, 0)),
        ],
        out_specs=pl.BlockSpec((2, blk, HALF), lambda i: (0, i, 0)),
        out_shape=jax.ShapeDtypeStruct((2, N_PAD, HALF), jnp.float32),
    )(x, W1.T, b1.reshape(1, NHID), W2.T, b2.reshape(1, NCLASS))


def _spmm_sc_body(h_hbm, row_hbm, col_hbm, w_hbm, out_hbm,
                  slab0, slab1, tmp_v, zero_v, g_v, col_v, row_v, w_vv):
    c = lax.axis_index("c")
    s = lax.axis_index("s")
    row0 = s * ROWS_PER_TILE

    def _zrow(r, _):
        zero_v[r, pl.ds(0, 16)] = jnp.zeros((16,), jnp.float32)
        zero_v[r, pl.ds(16, 16)] = jnp.zeros((16,), jnp.float32)
        return _
    lax.fori_loop(0, ROWS_PER_TILE, _zrow, None)

    pltpu.sync_copy(h_hbm.at[c, pl.ds(row0, ROWS_PER_TILE)], tmp_v)
    pltpu.sync_copy(tmp_v, slab0.at[pl.ds(row0, ROWS_PER_TILE)])

    def _make_edges(src, dst):
        def _edges(j, _):
            cid = s + NTILES * j

            @pl.when(cid < NCHUNKS)
            def _():
                base = cid * CHUNK
                pltpu.sync_copy(col_hbm.at[pl.ds(base, CHUNK)], col_v)
                pltpu.sync_copy(row_hbm.at[pl.ds(base, CHUNK)], row_v)
                pltpu.sync_copy(w_hbm.at[pl.ds(base, CHUNK)], w_vv)
                pltpu.sync_copy(src.at[col_v], g_v)

                def _scale(q, _c):
                    wv16 = w_vv[pl.ds(q * 16, 16)]
                    for l in range(16):
                        e = q * 16 + l
                        wv = wv16[l]
                        g_v[e, pl.ds(0, 16)] = g_v[e, pl.ds(0, 16)] * wv
                        g_v[e, pl.ds(16, 16)] = g_v[e, pl.ds(16, 16)] * wv
                    return _c
                lax.fori_loop(0, CHUNK // 16, _scale, None)

                pltpu.sync_copy(g_v, dst.at[row_v], add=True)
            return _
        return _edges

    for i in range(HOPS):
        src, dst = (slab0, slab1) if i % 2 == 0 else (slab1, slab0)
        pltpu.sync_copy(zero_v, dst.at[pl.ds(row0, ROWS_PER_TILE)])
        plsc.subcore_barrier()
        lax.fori_loop(0, CHUNKS_PER_TILE, _make_edges(src, dst), None)
        plsc.subcore_barrier()
        pltpu.sync_copy(dst.at[pl.ds(row0, ROWS_PER_TILE)], tmp_v)
        pltpu.sync_copy(tmp_v, out_hbm.at[i, c, pl.ds(row0, ROWS_PER_TILE)])


def _spmm_sc(h, row_f, col_f, w_f):
    mesh = plsc.VectorSubcoreMesh(core_axis_name="c", subcore_axis_name="s")
    kern = functools.partial(
        pl.kernel,
        mesh=mesh,
        compiler_params=pltpu.CompilerParams(use_tc_tiling_on_sc=False),
        out_type=jax.ShapeDtypeStruct((HOPS, 2, N_PAD, HALF), jnp.float32),
        scratch_types=[
            pltpu.VMEM_SHARED((N_PAD, HALF), jnp.float32),
            pltpu.VMEM_SHARED((N_PAD, HALF), jnp.float32),
            pltpu.VMEM((ROWS_PER_TILE, HALF), jnp.float32),
            pltpu.VMEM((ROWS_PER_TILE, HALF), jnp.float32),
            pltpu.VMEM((CHUNK, HALF), jnp.float32),
            pltpu.VMEM((CHUNK,), jnp.int32),
            pltpu.VMEM((CHUNK,), jnp.int32),
            pltpu.VMEM((CHUNK,), jnp.float32),
        ],
    )(_spmm_sc_body)
    return kern(h, row_f, col_f, w_f)


def _finish_body(dw_ref, slabs_ref, o_ref):
    y = None
    for i in range(HOPS):
        t = jnp.concatenate([slabs_ref[i, 0], slabs_ref[i, 1]], axis=1)
        t = dw_ref[i] * t
        y = t if y is None else y + t
    m = jnp.max(y, axis=1, keepdims=True)
    z = y - m
    o_ref[...] = z - jnp.log(jnp.sum(jnp.exp(z), axis=1, keepdims=True))


def _finish(slabs, dw):
    blk = 1000
    grid = N // blk
    return pl.pallas_call(
        _finish_body,
        grid=(grid,),
        in_specs=[
            pl.BlockSpec(memory_space=pltpu.SMEM),
            pl.BlockSpec((HOPS, 2, blk, HALF), lambda i: (0, 0, i, 0)),
        ],
        out_specs=pl.BlockSpec((blk, NCLASS), lambda i: (i, 0)),
        out_shape=jax.ShapeDtypeStruct((N, NCLASS), jnp.float32),
    )(dw, slabs)


@jax.jit
def kernel(x, edge_index, edge_weight, W1, b1, W2, b2, distance_weight):
    h = _mlp(x, W1, b1, W2, b2)
    slabs = _spmm_sc(h, edge_index[0], edge_index[1], edge_weight)
    return _finish(slabs, distance_weight)
